# Initial kernel scaffold; baseline (speedup 1.0000x reference)
import jax
import jax.numpy as jnp
from jax.experimental import pallas as pl


def kernel(x, edge_index, W1, b1, W2, b2, W3, b3):
    raise NotImplementedError("write your pallas kernel here")


# speedup vs baseline: 23.7004x; 23.7004x over previous
import functools

import jax
import jax.numpy as jnp
from jax import lax
from jax.experimental import pallas as pl
from jax.experimental.pallas import tpu as pltpu
from jax.experimental.pallas import tpu_sc as plsc

f32 = jnp.float32

N = 10000
D = 128
E = 320000

NC = 2
NS = 16
NW = NC * NS
EPW = E // NW
CHUNK = 80
NCHUNKS = EPW // CHUNK
ZR = 624
TAIL = N - NS * ZR
DEGW = 16

_mesh = plsc.VectorSubcoreMesh(core_axis_name="c", subcore_axis_name="s")


def _zero_zbuf(zbuf_v, width):
    z16 = jnp.zeros((16,), f32)
    for r in range(16):
        if width == 16:
            zbuf_v[r] = z16
        else:
            for c in range(width // 16):
                zbuf_v[r, pl.ds(c * 16, 16)] = z16


def _zero_acc_slice(zbuf_v, acc_sh, sid):
    zbase = sid * ZR

    def zbody(c, carry):
        pltpu.sync_copy(zbuf_v, acc_sh.at[pl.ds(zbase + c * 16, 16)])
        return carry

    lax.fori_loop(0, ZR // 16, zbody, 0)

    @pl.when(sid == NS - 1)
    def _ztail():
        pltpu.sync_copy(zbuf_v, acc_sh.at[pl.ds(NS * ZR, TAIL)])


def _copy_out_slice(acc_sh, out_hbm, cid, sid):
    zbase = sid * ZR
    pltpu.sync_copy(acc_sh.at[pl.ds(zbase, ZR)],
                    out_hbm.at[cid].at[pl.ds(zbase, ZR)])

    @pl.when(sid == NS - 1)
    def _ctail():
        pltpu.sync_copy(acc_sh.at[pl.ds(NS * ZR, TAIL)],
                        out_hbm.at[cid].at[pl.ds(NS * ZR, TAIL)])


def _sc_deg_body(dstr_hbm, deg_hbm, didx_v, ones_v, zbuf_v, deg_sh, sem):
    cid = lax.axis_index("c")
    sid = lax.axis_index("s")
    wid = sid * NC + cid
    one16 = jnp.ones((16,), f32)
    for r in range(CHUNK):
        ones_v[r] = one16
    _zero_zbuf(zbuf_v, DEGW)
    pltpu.sync_copy(dstr_hbm.at[wid], didx_v)
    _zero_acc_slice(zbuf_v, deg_sh, sid)
    plsc.subcore_barrier()

    K = 25

    def sbody(gidx, carry):
        cps = []
        for u in range(K):
            i = gidx * K + u
            cps.append(pltpu.async_copy(ones_v, deg_sh.at[didx_v.at[i]], sem,
                                        add=True))
        for cp in cps:
            cp.wait()
        return carry

    lax.fori_loop(0, NCHUNKS // K, sbody, 0)
    plsc.subcore_barrier()
    _copy_out_slice(deg_sh, deg_hbm, cid, sid)


_sc_deg = functools.partial(
    pl.kernel,
    out_type=jax.ShapeDtypeStruct((NC, N, DEGW), f32),
    mesh=_mesh,
    scratch_types=[
        pltpu.VMEM((NCHUNKS, CHUNK), jnp.int32),
        pltpu.VMEM((CHUNK, DEGW), f32),
        pltpu.VMEM((16, DEGW), f32),
        pltpu.VMEM_SHARED((N, DEGW), f32),
        pltpu.SemaphoreType.DMA,
    ],
)(_sc_deg_body)


NBUF = 2
NWIN = 5
WCH = 25


def _sc_agg_body(g_hbm, srcr_hbm, dstr_hbm, out_hbm,
                 sidx_v, didx_v, rows_v, zbuf_v, acc_sh, sem0, sem1):
    cid = lax.axis_index("c")
    sid = lax.axis_index("s")
    wid = sid * NC + cid
    sems = (sem0, sem1)
    _zero_zbuf(zbuf_v, D)
    _zero_acc_slice(zbuf_v, acc_sh, sid)
    plsc.subcore_barrier()

    def win(w, carry):
        pltpu.sync_copy(srcr_hbm.at[wid].at[w], sidx_v)
        pltpu.sync_copy(dstr_hbm.at[wid].at[w], didx_v)
        for b in range(NBUF):
            pltpu.async_copy(g_hbm.at[sidx_v.at[b]], rows_v.at[b], sems[b])

        def body(i, c2):
            for b in range(NBUF):
                t = i * NBUF + b
                pltpu.make_async_copy(g_hbm.at[sidx_v.at[t]], rows_v.at[b],
                                      sems[b]).wait()
                pltpu.sync_copy(rows_v.at[b], acc_sh.at[didx_v.at[t]],
                                add=True)

                @pl.when(t + NBUF < WCH)
                def _prefetch():
                    pltpu.async_copy(g_hbm.at[sidx_v.at[t + NBUF]],
                                     rows_v.at[b], sems[b])
            return c2

        lax.fori_loop(0, WCH // NBUF, body, 0)
        for t2 in range(WCH - WCH % NBUF, WCH):
            bb = t2 % NBUF
            pltpu.make_async_copy(g_hbm.at[sidx_v.at[t2]], rows_v.at[bb],
                                  sems[bb]).wait()
            pltpu.sync_copy(rows_v.at[bb], acc_sh.at[didx_v.at[t2]], add=True)
        return carry

    lax.fori_loop(0, NWIN, win, 0)
    plsc.subcore_barrier()
    _copy_out_slice(acc_sh, out_hbm, cid, sid)


_sc_agg = functools.partial(
    pl.kernel,
    out_type=jax.ShapeDtypeStruct((NC, N, D), f32),
    mesh=_mesh,
    scratch_types=[
        pltpu.VMEM((WCH, CHUNK), jnp.int32),
        pltpu.VMEM((WCH, CHUNK), jnp.int32),
        pltpu.VMEM((NBUF, CHUNK, D), f32),
        pltpu.VMEM((16, D), f32),
        pltpu.VMEM_SHARED((N, D), f32),
        pltpu.SemaphoreType.DMA,
        pltpu.SemaphoreType.DMA,
    ],
)(_sc_agg_body)


MB = 1000
GRID = N // MB


def _tc_first_body(d0_ref, d1_ref, x_ref, w_ref, g_ref, dinv_ref):
    deg = d0_ref[:, 0:1] + d1_ref[:, 0:1] + 1.0
    dinv = lax.rsqrt(deg)
    h = jnp.dot(x_ref[...], w_ref[...], preferred_element_type=f32)
    g_ref[...] = h * dinv
    dinv_ref[...] = dinv


_tc_first = pl.pallas_call(
    _tc_first_body,
    grid=(GRID,),
    in_specs=[
        pl.BlockSpec((MB, DEGW), lambda i: (i, 0)),
        pl.BlockSpec((MB, DEGW), lambda i: (i, 0)),
        pl.BlockSpec((MB, D), lambda i: (i, 0)),
        pl.BlockSpec((D, D), lambda i: (0, 0)),
    ],
    out_specs=[
        pl.BlockSpec((MB, D), lambda i: (i, 0)),
        pl.BlockSpec((MB, 1), lambda i: (i, 0)),
    ],
    out_shape=[
        jax.ShapeDtypeStruct((N, D), f32),
        jax.ShapeDtypeStruct((N, 1), f32),
    ],
)


def _tc_mid_body(s0_ref, s1_ref, g_ref, dinv_ref, b_ref, w_ref, gout_ref):
    dinv = dinv_ref[...]
    z = dinv * (s0_ref[...] + s1_ref[...] + g_ref[...]) + b_ref[...]
    r = jnp.maximum(z, 0.0)
    gout_ref[...] = jnp.dot(r, w_ref[...], preferred_element_type=f32) * dinv


_tc_mid = pl.pallas_call(
    _tc_mid_body,
    grid=(GRID,),
    in_specs=[
        pl.BlockSpec((MB, D), lambda i: (i, 0)),
        pl.BlockSpec((MB, D), lambda i: (i, 0)),
        pl.BlockSpec((MB, D), lambda i: (i, 0)),
        pl.BlockSpec((MB, 1), lambda i: (i, 0)),
        pl.BlockSpec((D,), lambda i: (0,)),
        pl.BlockSpec((D, D), lambda i: (0, 0)),
    ],
    out_specs=pl.BlockSpec((MB, D), lambda i: (i, 0)),
    out_shape=jax.ShapeDtypeStruct((N, D), f32),
)


def _tc_last_body(s0_ref, s1_ref, g_ref, dinv_ref, b_ref, out_ref):
    z = dinv_ref[...] * (s0_ref[...] + s1_ref[...] + g_ref[...]) + b_ref[...]
    m = jnp.max(z, axis=-1, keepdims=True)
    e = jnp.exp(z - m)
    lse = jnp.log(jnp.sum(e, axis=-1, keepdims=True))
    out_ref[...] = z - m - lse


_tc_last = pl.pallas_call(
    _tc_last_body,
    grid=(GRID,),
    in_specs=[
        pl.BlockSpec((MB, D), lambda i: (i, 0)),
        pl.BlockSpec((MB, D), lambda i: (i, 0)),
        pl.BlockSpec((MB, D), lambda i: (i, 0)),
        pl.BlockSpec((MB, 1), lambda i: (i, 0)),
        pl.BlockSpec((D,), lambda i: (0,)),
    ],
    out_specs=pl.BlockSpec((MB, D), lambda i: (i, 0)),
    out_shape=jax.ShapeDtypeStruct((N, D), f32),
)


def kernel(x, edge_index, W1, b1, W2, b2, W3, b3):
    src = edge_index[0]
    dst = edge_index[1]
    srcr = src.reshape(NW, NWIN, WCH, CHUNK)
    dstr = dst.reshape(NW, NWIN, WCH, CHUNK)

    degs = _sc_deg(dst.reshape(NW, NCHUNKS, CHUNK))

    g1, dinv = _tc_first(degs[0], degs[1], x, W1)
    s1 = _sc_agg(g1, srcr, dstr)
    g2 = _tc_mid(s1[0], s1[1], g1, dinv, b1, W2)
    s2 = _sc_agg(g2, srcr, dstr)
    g3 = _tc_mid(s2[0], s2[1], g2, dinv, b2, W3)
    s3 = _sc_agg(g3, srcr, dstr)
    return _tc_last(s3[0], s3[1], g3, dinv, b3)


# speedup vs baseline: 26.7168x; 1.1273x over previous
import functools

import jax
import jax.numpy as jnp
from jax import lax
from jax.experimental import pallas as pl
from jax.experimental.pallas import tpu as pltpu
from jax.experimental.pallas import tpu_sc as plsc

f32 = jnp.float32

N = 10000
D = 128
E = 320000

NC = 2
NS = 16
NW = NC * NS
EPW = E // NW
CHUNK = 80
NCHUNKS = EPW // CHUNK
ZR = 624
TAIL = N - NS * ZR
DEGW = 16

_mesh = plsc.VectorSubcoreMesh(core_axis_name="c", subcore_axis_name="s")


def _zero_zbuf(zbuf_v, width):
    z16 = jnp.zeros((16,), f32)
    for r in range(16):
        if width == 16:
            zbuf_v[r] = z16
        else:
            for c in range(width // 16):
                zbuf_v[r, pl.ds(c * 16, 16)] = z16


def _zero_acc_slice(zbuf_v, acc_sh, sid):
    zbase = sid * ZR

    def zbody(c, carry):
        pltpu.sync_copy(zbuf_v, acc_sh.at[pl.ds(zbase + c * 16, 16)])
        return carry

    lax.fori_loop(0, ZR // 16, zbody, 0)

    @pl.when(sid == NS - 1)
    def _ztail():
        pltpu.sync_copy(zbuf_v, acc_sh.at[pl.ds(NS * ZR, TAIL)])


def _copy_out_slice(acc_sh, out_hbm, cid, sid):
    zbase = sid * ZR
    pltpu.sync_copy(acc_sh.at[pl.ds(zbase, ZR)],
                    out_hbm.at[cid].at[pl.ds(zbase, ZR)])

    @pl.when(sid == NS - 1)
    def _ctail():
        pltpu.sync_copy(acc_sh.at[pl.ds(NS * ZR, TAIL)],
                        out_hbm.at[cid].at[pl.ds(NS * ZR, TAIL)])


def _sc_deg_body(dstr_hbm, deg_hbm, didx_v, ones_v, zbuf_v, deg_sh, sem):
    cid = lax.axis_index("c")
    sid = lax.axis_index("s")
    wid = sid * NC + cid
    one16 = jnp.ones((16,), f32)
    for r in range(CHUNK):
        ones_v[r] = one16
    _zero_zbuf(zbuf_v, DEGW)
    pltpu.sync_copy(dstr_hbm.at[wid], didx_v)
    _zero_acc_slice(zbuf_v, deg_sh, sid)
    plsc.subcore_barrier()

    K = 25

    def sbody(gidx, carry):
        cps = []
        for u in range(K):
            i = gidx * K + u
            cps.append(pltpu.async_copy(ones_v, deg_sh.at[didx_v.at[i]], sem,
                                        add=True))
        for cp in cps:
            cp.wait()
        return carry

    lax.fori_loop(0, NCHUNKS // K, sbody, 0)
    plsc.subcore_barrier()
    _copy_out_slice(deg_sh, deg_hbm, cid, sid)


_sc_deg = functools.partial(
    pl.kernel,
    out_type=jax.ShapeDtypeStruct((NC, N, DEGW), f32),
    mesh=_mesh,
    scratch_types=[
        pltpu.VMEM((NCHUNKS, CHUNK), jnp.int32),
        pltpu.VMEM((CHUNK, DEGW), f32),
        pltpu.VMEM((16, DEGW), f32),
        pltpu.VMEM_SHARED((N, DEGW), f32),
        pltpu.SemaphoreType.DMA,
    ],
)(_sc_deg_body)


NBUF = 3
NWIN = 5
WCH = 25


def _sc_agg_body(g_hbm, srcr_hbm, dstr_hbm, out_hbm,
                 sidx_v, didx_v, rows_v, zbuf_v, acc_sh,
                 g0, g1, g2, s0, s1, s2):
    cid = lax.axis_index("c")
    sid = lax.axis_index("s")
    wid = sid * NC + cid
    gsems = (g0, g1, g2)
    ssems = (s0, s1, s2)
    _zero_zbuf(zbuf_v, D)
    _zero_acc_slice(zbuf_v, acc_sh, sid)
    plsc.subcore_barrier()

    def win(w, carry):
        pltpu.sync_copy(srcr_hbm.at[wid].at[w], sidx_v)
        pltpu.sync_copy(dstr_hbm.at[wid].at[w], didx_v)
        for b in range(NBUF):
            pltpu.async_copy(g_hbm.at[sidx_v.at[b]], rows_v.at[b], gsems[b])

        def body(i, c2):
            for b in range(NBUF):
                t = i * NBUF + b
                pltpu.make_async_copy(g_hbm.at[sidx_v.at[t]], rows_v.at[b],
                                      gsems[b]).wait()
                pltpu.async_copy(rows_v.at[b], acc_sh.at[didx_v.at[t]],
                                 ssems[b], add=True)

                @pl.when(t + NBUF < WCH)
                def _next():
                    pltpu.make_async_copy(rows_v.at[b],
                                          acc_sh.at[didx_v.at[t]],
                                          ssems[b]).wait()
                    pltpu.async_copy(g_hbm.at[sidx_v.at[t + NBUF]],
                                     rows_v.at[b], gsems[b])
            return c2

        lax.fori_loop(0, WCH // NBUF, body, 0)
        for t2 in range((WCH // NBUF) * NBUF, WCH):
            bb = t2 % NBUF
            pltpu.make_async_copy(g_hbm.at[sidx_v.at[t2]], rows_v.at[bb],
                                  gsems[bb]).wait()
            pltpu.async_copy(rows_v.at[bb], acc_sh.at[didx_v.at[t2]],
                             ssems[bb], add=True)
        for t2 in range(WCH - NBUF, WCH):
            bb = t2 % NBUF
            pltpu.make_async_copy(rows_v.at[bb], acc_sh.at[didx_v.at[t2]],
                                  ssems[bb]).wait()
        return carry

    lax.fori_loop(0, NWIN, win, 0)
    plsc.subcore_barrier()
    _copy_out_slice(acc_sh, out_hbm, cid, sid)


_sc_agg = functools.partial(
    pl.kernel,
    out_type=jax.ShapeDtypeStruct((NC, N, D), f32),
    mesh=_mesh,
    scratch_types=[
        pltpu.VMEM((WCH, CHUNK), jnp.int32),
        pltpu.VMEM((WCH, CHUNK), jnp.int32),
        pltpu.VMEM((NBUF, CHUNK, D), f32),
        pltpu.VMEM((16, D), f32),
        pltpu.VMEM_SHARED((N, D), f32),
        pltpu.SemaphoreType.DMA,
        pltpu.SemaphoreType.DMA,
        pltpu.SemaphoreType.DMA,
        pltpu.SemaphoreType.DMA,
        pltpu.SemaphoreType.DMA,
        pltpu.SemaphoreType.DMA,
    ],
)(_sc_agg_body)


MB = 1000
GRID = N // MB


def _tc_first_body(d0_ref, d1_ref, x_ref, w_ref, g_ref, dinv_ref):
    deg = d0_ref[:, 0:1] + d1_ref[:, 0:1] + 1.0
    dinv = lax.rsqrt(deg)
    h = jnp.dot(x_ref[...], w_ref[...], preferred_element_type=f32)
    g_ref[...] = h * dinv
    dinv_ref[...] = dinv


_tc_first = pl.pallas_call(
    _tc_first_body,
    grid=(GRID,),
    in_specs=[
        pl.BlockSpec((MB, DEGW), lambda i: (i, 0)),
        pl.BlockSpec((MB, DEGW), lambda i: (i, 0)),
        pl.BlockSpec((MB, D), lambda i: (i, 0)),
        pl.BlockSpec((D, D), lambda i: (0, 0)),
    ],
    out_specs=[
        pl.BlockSpec((MB, D), lambda i: (i, 0)),
        pl.BlockSpec((MB, 1), lambda i: (i, 0)),
    ],
    out_shape=[
        jax.ShapeDtypeStruct((N, D), f32),
        jax.ShapeDtypeStruct((N, 1), f32),
    ],
)


def _tc_mid_body(s0_ref, s1_ref, g_ref, dinv_ref, b_ref, w_ref, gout_ref):
    dinv = dinv_ref[...]
    z = dinv * (s0_ref[...] + s1_ref[...] + g_ref[...]) + b_ref[...]
    r = jnp.maximum(z, 0.0)
    gout_ref[...] = jnp.dot(r, w_ref[...], preferred_element_type=f32) * dinv


_tc_mid = pl.pallas_call(
    _tc_mid_body,
    grid=(GRID,),
    in_specs=[
        pl.BlockSpec((MB, D), lambda i: (i, 0)),
        pl.BlockSpec((MB, D), lambda i: (i, 0)),
        pl.BlockSpec((MB, D), lambda i: (i, 0)),
        pl.BlockSpec((MB, 1), lambda i: (i, 0)),
        pl.BlockSpec((D,), lambda i: (0,)),
        pl.BlockSpec((D, D), lambda i: (0, 0)),
    ],
    out_specs=pl.BlockSpec((MB, D), lambda i: (i, 0)),
    out_shape=jax.ShapeDtypeStruct((N, D), f32),
)


def _tc_last_body(s0_ref, s1_ref, g_ref, dinv_ref, b_ref, out_ref):
    z = dinv_ref[...] * (s0_ref[...] + s1_ref[...] + g_ref[...]) + b_ref[...]
    m = jnp.max(z, axis=-1, keepdims=True)
    e = jnp.exp(z - m)
    lse = jnp.log(jnp.sum(e, axis=-1, keepdims=True))
    out_ref[...] = z - m - lse


_tc_last = pl.pallas_call(
    _tc_last_body,
    grid=(GRID,),
    in_specs=[
        pl.BlockSpec((MB, D), lambda i: (i, 0)),
        pl.BlockSpec((MB, D), lambda i: (i, 0)),
        pl.BlockSpec((MB, D), lambda i: (i, 0)),
        pl.BlockSpec((MB, 1), lambda i: (i, 0)),
        pl.BlockSpec((D,), lambda i: (0,)),
    ],
    out_specs=pl.BlockSpec((MB, D), lambda i: (i, 0)),
    out_shape=jax.ShapeDtypeStruct((N, D), f32),
)


def kernel(x, edge_index, W1, b1, W2, b2, W3, b3):
    src = edge_index[0]
    dst = edge_index[1]
    srcr = src.reshape(NW, NWIN, WCH, CHUNK)
    dstr = dst.reshape(NW, NWIN, WCH, CHUNK)

    degs = _sc_deg(dst.reshape(NW, NCHUNKS, CHUNK))

    g1, dinv = _tc_first(degs[0], degs[1], x, W1)
    s1 = _sc_agg(g1, srcr, dstr)
    g2 = _tc_mid(s1[0], s1[1], g1, dinv, b1, W2)
    s2 = _sc_agg(g2, srcr, dstr)
    g3 = _tc_mid(s2[0], s2[1], g2, dinv, b2, W3)
    s3 = _sc_agg(g3, srcr, dstr)
    return _tc_last(s3[0], s3[1], g3, dinv, b3)


# speedup vs baseline: 28.3978x; 1.0629x over previous
import functools

import jax
import jax.numpy as jnp
from jax import lax
from jax.experimental import pallas as pl
from jax.experimental.pallas import tpu as pltpu
from jax.experimental.pallas import tpu_sc as plsc

f32 = jnp.float32

N = 10000
D = 128
E = 320000

NC = 2
NS = 16
NW = NC * NS
EPW = E // NW
CHUNK = 80
NCHUNKS = EPW // CHUNK
ZR = 624
TAIL = N - NS * ZR
DEGW = 16

_mesh = plsc.VectorSubcoreMesh(core_axis_name="c", subcore_axis_name="s")


def _zero_zbuf(zbuf_v, width):
    z16 = jnp.zeros((16,), f32)
    for r in range(16):
        if width == 16:
            zbuf_v[r] = z16
        else:
            for c in range(width // 16):
                zbuf_v[r, pl.ds(c * 16, 16)] = z16


def _zero_acc_slice(zbuf_v, acc_sh, sid):
    zbase = sid * ZR

    def zbody(c, carry):
        pltpu.sync_copy(zbuf_v, acc_sh.at[pl.ds(zbase + c * 16, 16)])
        return carry

    lax.fori_loop(0, ZR // 16, zbody, 0)

    @pl.when(sid == NS - 1)
    def _ztail():
        pltpu.sync_copy(zbuf_v, acc_sh.at[pl.ds(NS * ZR, TAIL)])


def _copy_out_slice(acc_sh, out_hbm, cid, sid):
    zbase = sid * ZR
    pltpu.sync_copy(acc_sh.at[pl.ds(zbase, ZR)],
                    out_hbm.at[cid].at[pl.ds(zbase, ZR)])

    @pl.when(sid == NS - 1)
    def _ctail():
        pltpu.sync_copy(acc_sh.at[pl.ds(NS * ZR, TAIL)],
                        out_hbm.at[cid].at[pl.ds(NS * ZR, TAIL)])


def _sc_deg_body(dstr_hbm, deg_hbm, didx_v, ones_v, zbuf_v, deg_sh, sem):
    cid = lax.axis_index("c")
    sid = lax.axis_index("s")
    wid = sid * NC + cid
    one16 = jnp.ones((16,), f32)
    for r in range(CHUNK):
        ones_v[r] = one16
    _zero_zbuf(zbuf_v, DEGW)
    pltpu.sync_copy(dstr_hbm.at[wid], didx_v)
    _zero_acc_slice(zbuf_v, deg_sh, sid)
    plsc.subcore_barrier()

    K = 25

    def sbody(gidx, carry):
        cps = []
        for u in range(K):
            i = gidx * K + u
            cps.append(pltpu.async_copy(ones_v, deg_sh.at[didx_v.at[i]], sem,
                                        add=True))
        for cp in cps:
            cp.wait()
        return carry

    lax.fori_loop(0, NCHUNKS // K, sbody, 0)
    plsc.subcore_barrier()
    _copy_out_slice(deg_sh, deg_hbm, cid, sid)


_sc_deg = functools.partial(
    pl.kernel,
    out_type=jax.ShapeDtypeStruct((NC, N, DEGW), f32),
    mesh=_mesh,
    scratch_types=[
        pltpu.VMEM((NCHUNKS, CHUNK), jnp.int32),
        pltpu.VMEM((CHUNK, DEGW), f32),
        pltpu.VMEM((16, DEGW), f32),
        pltpu.VMEM_SHARED((N, DEGW), f32),
        pltpu.SemaphoreType.DMA,
    ],
)(_sc_deg_body)


NBUF = 3
NWIN = 5
WCH = 25


def _sc_agg_body(g_hbm, srcr_hbm, dstr_hbm, out_hbm,
                 sidx_v, didx_v, rows_v, zbuf_v, acc_sh,
                 g0, g1, g2, s0, s1, s2):
    cid = lax.axis_index("c")
    sid = lax.axis_index("s")
    wid = sid * NC + cid
    gsems = (g0, g1, g2)
    ssems = (s0, s1, s2)
    _zero_zbuf(zbuf_v, D)
    _zero_acc_slice(zbuf_v, acc_sh, sid)
    plsc.subcore_barrier()

    def win(w, carry):
        pltpu.sync_copy(srcr_hbm.at[wid].at[w], sidx_v)
        pltpu.sync_copy(dstr_hbm.at[wid].at[w], didx_v)
        for b in range(NBUF):
            pltpu.async_copy(g_hbm.at[sidx_v.at[b]], rows_v.at[b], gsems[b])

        def body(i, c2):
            for b in range(NBUF):
                t = i * NBUF + b
                pltpu.make_async_copy(g_hbm.at[sidx_v.at[t]], rows_v.at[b],
                                      gsems[b]).wait()
                pltpu.async_copy(rows_v.at[b], acc_sh.at[didx_v.at[t]],
                                 ssems[b], add=True)

                @pl.when(t + NBUF < WCH)
                def _next():
                    pltpu.make_async_copy(rows_v.at[b],
                                          acc_sh.at[didx_v.at[t]],
                                          ssems[b]).wait()
                    pltpu.async_copy(g_hbm.at[sidx_v.at[t + NBUF]],
                                     rows_v.at[b], gsems[b])
            return c2

        lax.fori_loop(0, WCH // NBUF, body, 0)
        for t2 in range((WCH // NBUF) * NBUF, WCH):
            bb = t2 % NBUF
            pltpu.make_async_copy(g_hbm.at[sidx_v.at[t2]], rows_v.at[bb],
                                  gsems[bb]).wait()
            pltpu.async_copy(rows_v.at[bb], acc_sh.at[didx_v.at[t2]],
                             ssems[bb], add=True)
        for t2 in range(WCH - NBUF, WCH):
            bb = t2 % NBUF
            pltpu.make_async_copy(rows_v.at[bb], acc_sh.at[didx_v.at[t2]],
                                  ssems[bb]).wait()
        return carry

    lax.fori_loop(0, NWIN, win, 0)
    plsc.subcore_barrier()
    _copy_out_slice(acc_sh, out_hbm, cid, sid)


_sc_agg = functools.partial(
    pl.kernel,
    out_type=jax.ShapeDtypeStruct((NC, N, D), f32),
    mesh=_mesh,
    scratch_types=[
        pltpu.VMEM((WCH, CHUNK), jnp.int32),
        pltpu.VMEM((WCH, CHUNK), jnp.int32),
        pltpu.VMEM((NBUF, CHUNK, D), f32),
        pltpu.VMEM((16, D), f32),
        pltpu.VMEM_SHARED((N, D), f32),
        pltpu.SemaphoreType.DMA,
        pltpu.SemaphoreType.DMA,
        pltpu.SemaphoreType.DMA,
        pltpu.SemaphoreType.DMA,
        pltpu.SemaphoreType.DMA,
        pltpu.SemaphoreType.DMA,
    ],
)(_sc_agg_body)


MB = 1000
GRID = N // MB


def _tc_first_body(degs_ref, x_ref, w_ref, g_ref, dinv_ref):
    deg = degs_ref[0, :, 0:1] + degs_ref[1, :, 0:1] + 1.0
    dinv = lax.rsqrt(deg)
    h = jnp.dot(x_ref[...], w_ref[...], preferred_element_type=f32)
    g_ref[...] = h * dinv
    dinv_ref[...] = dinv


_tc_first = pl.pallas_call(
    _tc_first_body,
    grid=(GRID,),
    in_specs=[
        pl.BlockSpec((2, MB, DEGW), lambda i: (0, i, 0)),
        pl.BlockSpec((MB, D), lambda i: (i, 0)),
        pl.BlockSpec((D, D), lambda i: (0, 0)),
    ],
    out_specs=[
        pl.BlockSpec((MB, D), lambda i: (i, 0)),
        pl.BlockSpec((MB, 1), lambda i: (i, 0)),
    ],
    out_shape=[
        jax.ShapeDtypeStruct((N, D), f32),
        jax.ShapeDtypeStruct((N, 1), f32),
    ],
)


def _tc_mid_body(s_ref, g_ref, dinv_ref, b_ref, w_ref, gout_ref):
    dinv = dinv_ref[...]
    z = dinv * (s_ref[0] + s_ref[1] + g_ref[...]) + b_ref[...]
    r = jnp.maximum(z, 0.0)
    gout_ref[...] = jnp.dot(r, w_ref[...], preferred_element_type=f32) * dinv


_tc_mid = pl.pallas_call(
    _tc_mid_body,
    grid=(GRID,),
    in_specs=[
        pl.BlockSpec((2, MB, D), lambda i: (0, i, 0)),
        pl.BlockSpec((MB, D), lambda i: (i, 0)),
        pl.BlockSpec((MB, 1), lambda i: (i, 0)),
        pl.BlockSpec((D,), lambda i: (0,)),
        pl.BlockSpec((D, D), lambda i: (0, 0)),
    ],
    out_specs=pl.BlockSpec((MB, D), lambda i: (i, 0)),
    out_shape=jax.ShapeDtypeStruct((N, D), f32),
)


def _tc_last_body(s_ref, g_ref, dinv_ref, b_ref, out_ref):
    z = dinv_ref[...] * (s_ref[0] + s_ref[1] + g_ref[...]) + b_ref[...]
    m = jnp.max(z, axis=-1, keepdims=True)
    e = jnp.exp(z - m)
    lse = jnp.log(jnp.sum(e, axis=-1, keepdims=True))
    out_ref[...] = z - m - lse


_tc_last = pl.pallas_call(
    _tc_last_body,
    grid=(GRID,),
    in_specs=[
        pl.BlockSpec((2, MB, D), lambda i: (0, i, 0)),
        pl.BlockSpec((MB, D), lambda i: (i, 0)),
        pl.BlockSpec((MB, 1), lambda i: (i, 0)),
        pl.BlockSpec((D,), lambda i: (0,)),
    ],
    out_specs=pl.BlockSpec((MB, D), lambda i: (i, 0)),
    out_shape=jax.ShapeDtypeStruct((N, D), f32),
)


def kernel(x, edge_index, W1, b1, W2, b2, W3, b3):
    src = edge_index[0]
    dst = edge_index[1]
    srcr = src.reshape(NW, NWIN, WCH, CHUNK)
    dstr = dst.reshape(NW, NWIN, WCH, CHUNK)

    degs = _sc_deg(dst.reshape(NW, NCHUNKS, CHUNK))

    g1, dinv = _tc_first(degs, x, W1)
    s1 = _sc_agg(g1, srcr, dstr)
    g2 = _tc_mid(s1, g1, dinv, b1, W2)
    s2 = _sc_agg(g2, srcr, dstr)
    g3 = _tc_mid(s2, g2, dinv, b2, W3)
    s3 = _sc_agg(g3, srcr, dstr)
    return _tc_last(s3, g3, dinv, b3)


# speedup vs baseline: 30.3416x; 1.0684x over previous
import functools

import jax
import jax.numpy as jnp
from jax import lax
from jax.experimental import pallas as pl
from jax.experimental.pallas import tpu as pltpu
from jax.experimental.pallas import tpu_sc as plsc

f32 = jnp.float32

N = 10000
D = 128
E = 320000

NC = 2
NS = 16
NW = NC * NS
EPW = E // NW
CHUNK = 80
NCHUNKS = EPW // CHUNK
ZR = 624
TAIL = N - NS * ZR
DEGW = 16

_mesh = plsc.VectorSubcoreMesh(core_axis_name="c", subcore_axis_name="s")


def _zero_zbuf(zbuf_v, width):
    z16 = jnp.zeros((16,), f32)
    for r in range(16):
        if width == 16:
            zbuf_v[r] = z16
        else:
            for c in range(width // 16):
                zbuf_v[r, pl.ds(c * 16, 16)] = z16


def _zero_acc_slice(zbuf_v, acc_sh, sid, zsem):
    zbase = sid * ZR

    def zbody(c, carry):
        pltpu.async_copy(zbuf_v, acc_sh.at[pl.ds(zbase + c * 16, 16)], zsem)
        return carry

    lax.fori_loop(0, ZR // 16, zbody, 0)

    @pl.when(sid == NS - 1)
    def _ztail():
        pltpu.async_copy(zbuf_v, acc_sh.at[pl.ds(NS * ZR, TAIL)], zsem)

    def zdrain(c, carry):
        pltpu.make_async_copy(zbuf_v, acc_sh.at[pl.ds(zbase, 16)], zsem).wait()
        return carry

    lax.fori_loop(0, ZR // 16, zdrain, 0)

    @pl.when(sid == NS - 1)
    def _zdtail():
        pltpu.make_async_copy(zbuf_v, acc_sh.at[pl.ds(zbase, TAIL)],
                              zsem).wait()


def _copy_out_slice(acc_sh, out_hbm, cid, sid):
    zbase = sid * ZR
    pltpu.sync_copy(acc_sh.at[pl.ds(zbase, ZR)],
                    out_hbm.at[cid].at[pl.ds(zbase, ZR)])

    @pl.when(sid == NS - 1)
    def _ctail():
        pltpu.sync_copy(acc_sh.at[pl.ds(NS * ZR, TAIL)],
                        out_hbm.at[cid].at[pl.ds(NS * ZR, TAIL)])


def _sc_deg_body(dstr_hbm, deg_hbm, didx_v, ones_v, zbuf_v, deg_sh, sem):
    cid = lax.axis_index("c")
    sid = lax.axis_index("s")
    wid = sid * NC + cid
    one16 = jnp.ones((16,), f32)
    for r in range(CHUNK):
        ones_v[r] = one16
    _zero_zbuf(zbuf_v, DEGW)
    pltpu.sync_copy(dstr_hbm.at[wid], didx_v)
    _zero_acc_slice(zbuf_v, deg_sh, sid, sem)
    plsc.subcore_barrier()

    K = 25

    def sbody(gidx, carry):
        cps = []
        for u in range(K):
            i = gidx * K + u
            cps.append(pltpu.async_copy(ones_v, deg_sh.at[didx_v.at[i]], sem,
                                        add=True))
        for cp in cps:
            cp.wait()
        return carry

    lax.fori_loop(0, NCHUNKS // K, sbody, 0)
    plsc.subcore_barrier()
    _copy_out_slice(deg_sh, deg_hbm, cid, sid)


_sc_deg = functools.partial(
    pl.kernel,
    out_type=jax.ShapeDtypeStruct((NC, N, DEGW), f32),
    mesh=_mesh,
    scratch_types=[
        pltpu.VMEM((NCHUNKS, CHUNK), jnp.int32),
        pltpu.VMEM((CHUNK, DEGW), f32),
        pltpu.VMEM((16, DEGW), f32),
        pltpu.VMEM_SHARED((N, DEGW), f32),
        pltpu.SemaphoreType.DMA,
    ],
)(_sc_deg_body)


NBUF = 3
NWIN = 5
WCH = 25


def _sc_agg_body(g_hbm, srcr_hbm, dstr_hbm, out_hbm,
                 sidx_v, didx_v, rows_v, zbuf_v, acc_sh,
                 g0, g1, g2, s0, s1, s2, isem):
    cid = lax.axis_index("c")
    sid = lax.axis_index("s")
    wid = sid * NC + cid
    gsems = (g0, g1, g2)
    ssems = (s0, s1, s2)
    _zero_zbuf(zbuf_v, D)
    pltpu.async_copy(srcr_hbm.at[wid].at[0], sidx_v.at[0], isem)
    pltpu.async_copy(dstr_hbm.at[wid].at[0], didx_v.at[0], isem)
    _zero_acc_slice(zbuf_v, acc_sh, sid, g0)
    plsc.subcore_barrier()

    def win(w, carry):
        ws = lax.rem(w, 2)
        sx = sidx_v.at[ws]
        dx = didx_v.at[ws]
        pltpu.make_async_copy(srcr_hbm.at[wid].at[w], sx, isem).wait()
        pltpu.make_async_copy(dstr_hbm.at[wid].at[w], dx, isem).wait()

        @pl.when(w + 1 < NWIN)
        def _pfw():
            pltpu.async_copy(srcr_hbm.at[wid].at[w + 1], sidx_v.at[1 - ws],
                             isem)
            pltpu.async_copy(dstr_hbm.at[wid].at[w + 1], didx_v.at[1 - ws],
                             isem)

        for b in range(NBUF):
            pltpu.async_copy(g_hbm.at[sx.at[b]], rows_v.at[b], gsems[b])

        def body(i, c2):
            for b in range(NBUF):
                t = i * NBUF + b
                pltpu.make_async_copy(g_hbm.at[sx.at[t]], rows_v.at[b],
                                      gsems[b]).wait()
                pltpu.async_copy(rows_v.at[b], acc_sh.at[dx.at[t]],
                                 ssems[b], add=True)

                @pl.when(t + NBUF < WCH)
                def _next():
                    pltpu.make_async_copy(rows_v.at[b],
                                          acc_sh.at[dx.at[t]],
                                          ssems[b]).wait()
                    pltpu.async_copy(g_hbm.at[sx.at[t + NBUF]],
                                     rows_v.at[b], gsems[b])
            return c2

        lax.fori_loop(0, WCH // NBUF, body, 0)
        for t2 in range((WCH // NBUF) * NBUF, WCH):
            bb = t2 % NBUF
            pltpu.make_async_copy(g_hbm.at[sx.at[t2]], rows_v.at[bb],
                                  gsems[bb]).wait()
            pltpu.async_copy(rows_v.at[bb], acc_sh.at[dx.at[t2]],
                             ssems[bb], add=True)
        for t2 in range(WCH - NBUF, WCH):
            bb = t2 % NBUF
            pltpu.make_async_copy(rows_v.at[bb], acc_sh.at[dx.at[t2]],
                                  ssems[bb]).wait()
        return carry

    lax.fori_loop(0, NWIN, win, 0)
    plsc.subcore_barrier()
    _copy_out_slice(acc_sh, out_hbm, cid, sid)


_sc_agg = functools.partial(
    pl.kernel,
    out_type=jax.ShapeDtypeStruct((NC, N, D), f32),
    mesh=_mesh,
    scratch_types=[
        pltpu.VMEM((2, WCH, CHUNK), jnp.int32),
        pltpu.VMEM((2, WCH, CHUNK), jnp.int32),
        pltpu.VMEM((NBUF, CHUNK, D), f32),
        pltpu.VMEM((16, D), f32),
        pltpu.VMEM_SHARED((N, D), f32),
        pltpu.SemaphoreType.DMA,
        pltpu.SemaphoreType.DMA,
        pltpu.SemaphoreType.DMA,
        pltpu.SemaphoreType.DMA,
        pltpu.SemaphoreType.DMA,
        pltpu.SemaphoreType.DMA,
        pltpu.SemaphoreType.DMA,
    ],
)(_sc_agg_body)


MB = 1000
GRID = N // MB


def _tc_first_body(degs_ref, x_ref, w_ref, g_ref, dinv_ref):
    deg = degs_ref[0, :, 0:1] + degs_ref[1, :, 0:1] + 1.0
    dinv = lax.rsqrt(deg)
    h = jnp.dot(x_ref[...], w_ref[...], preferred_element_type=f32)
    g_ref[...] = h * dinv
    dinv_ref[...] = dinv


_tc_first = pl.pallas_call(
    _tc_first_body,
    grid=(GRID,),
    in_specs=[
        pl.BlockSpec((2, MB, DEGW), lambda i: (0, i, 0)),
        pl.BlockSpec((MB, D), lambda i: (i, 0)),
        pl.BlockSpec((D, D), lambda i: (0, 0)),
    ],
    out_specs=[
        pl.BlockSpec((MB, D), lambda i: (i, 0)),
        pl.BlockSpec((MB, 1), lambda i: (i, 0)),
    ],
    out_shape=[
        jax.ShapeDtypeStruct((N, D), f32),
        jax.ShapeDtypeStruct((N, 1), f32),
    ],
)


def _tc_mid_body(s_ref, g_ref, dinv_ref, b_ref, w_ref, gout_ref):
    dinv = dinv_ref[...]
    z = dinv * (s_ref[0] + s_ref[1] + g_ref[...]) + b_ref[...]
    r = jnp.maximum(z, 0.0)
    gout_ref[...] = jnp.dot(r, w_ref[...], preferred_element_type=f32) * dinv


_tc_mid = pl.pallas_call(
    _tc_mid_body,
    grid=(GRID,),
    in_specs=[
        pl.BlockSpec((2, MB, D), lambda i: (0, i, 0)),
        pl.BlockSpec((MB, D), lambda i: (i, 0)),
        pl.BlockSpec((MB, 1), lambda i: (i, 0)),
        pl.BlockSpec((D,), lambda i: (0,)),
        pl.BlockSpec((D, D), lambda i: (0, 0)),
    ],
    out_specs=pl.BlockSpec((MB, D), lambda i: (i, 0)),
    out_shape=jax.ShapeDtypeStruct((N, D), f32),
)


def _tc_last_body(s_ref, g_ref, dinv_ref, b_ref, out_ref):
    z = dinv_ref[...] * (s_ref[0] + s_ref[1] + g_ref[...]) + b_ref[...]
    m = jnp.max(z, axis=-1, keepdims=True)
    e = jnp.exp(z - m)
    lse = jnp.log(jnp.sum(e, axis=-1, keepdims=True))
    out_ref[...] = z - m - lse


_tc_last = pl.pallas_call(
    _tc_last_body,
    grid=(GRID,),
    in_specs=[
        pl.BlockSpec((2, MB, D), lambda i: (0, i, 0)),
        pl.BlockSpec((MB, D), lambda i: (i, 0)),
        pl.BlockSpec((MB, 1), lambda i: (i, 0)),
        pl.BlockSpec((D,), lambda i: (0,)),
    ],
    out_specs=pl.BlockSpec((MB, D), lambda i: (i, 0)),
    out_shape=jax.ShapeDtypeStruct((N, D), f32),
)


def kernel(x, edge_index, W1, b1, W2, b2, W3, b3):
    src = edge_index[0]
    dst = edge_index[1]
    srcr = src.reshape(NW, NWIN, WCH, CHUNK)
    dstr = dst.reshape(NW, NWIN, WCH, CHUNK)

    degs = _sc_deg(dst.reshape(NW, NCHUNKS, CHUNK))

    g1, dinv = _tc_first(degs, x, W1)
    s1 = _sc_agg(g1, srcr, dstr)
    g2 = _tc_mid(s1, g1, dinv, b1, W2)
    s2 = _sc_agg(g2, srcr, dstr)
    g3 = _tc_mid(s2, g2, dinv, b2, W3)
    s3 = _sc_agg(g3, srcr, dstr)
    return _tc_last(s3, g3, dinv, b3)


# speedup vs baseline: 30.4033x; 1.0020x over previous
import functools

import jax
import jax.numpy as jnp
from jax import lax
from jax.experimental import pallas as pl
from jax.experimental.pallas import tpu as pltpu
from jax.experimental.pallas import tpu_sc as plsc

f32 = jnp.float32

N = 10000
D = 128
E = 320000

NC = 2
NS = 16
NW = NC * NS
EPW = E // NW
CHUNK = 80
NCHUNKS = EPW // CHUNK
ZR = 624
TAIL = N - NS * ZR
DEGW = 16

_mesh = plsc.VectorSubcoreMesh(core_axis_name="c", subcore_axis_name="s")


def _zero_zbuf(zbuf_v, width):
    z16 = jnp.zeros((16,), f32)
    for r in range(16):
        if width == 16:
            zbuf_v[r] = z16
        else:
            for c in range(width // 16):
                zbuf_v[r, pl.ds(c * 16, 16)] = z16


def _zero_acc_slice(zbuf_v, acc_sh, sid, zsem):
    zbase = sid * ZR

    def zbody(c, carry):
        pltpu.async_copy(zbuf_v, acc_sh.at[pl.ds(zbase + c * 16, 16)], zsem)
        return carry

    lax.fori_loop(0, ZR // 16, zbody, 0)

    @pl.when(sid == NS - 1)
    def _ztail():
        pltpu.async_copy(zbuf_v, acc_sh.at[pl.ds(NS * ZR, TAIL)], zsem)

    def zdrain(c, carry):
        pltpu.make_async_copy(zbuf_v, acc_sh.at[pl.ds(zbase, 16)], zsem).wait()
        return carry

    lax.fori_loop(0, ZR // 16, zdrain, 0)

    @pl.when(sid == NS - 1)
    def _zdtail():
        pltpu.make_async_copy(zbuf_v, acc_sh.at[pl.ds(zbase, TAIL)],
                              zsem).wait()


def _copy_out_slice(acc_sh, out_hbm, cid, sid):
    zbase = sid * ZR
    pltpu.sync_copy(acc_sh.at[pl.ds(zbase, ZR)],
                    out_hbm.at[cid].at[pl.ds(zbase, ZR)])

    @pl.when(sid == NS - 1)
    def _ctail():
        pltpu.sync_copy(acc_sh.at[pl.ds(NS * ZR, TAIL)],
                        out_hbm.at[cid].at[pl.ds(NS * ZR, TAIL)])


def _sc_deg_body(dstr_hbm, deg_hbm, didx_v, ones_v, zbuf_v, deg_sh, sem):
    cid = lax.axis_index("c")
    sid = lax.axis_index("s")
    wid = sid * NC + cid
    one16 = jnp.ones((16,), f32)
    for r in range(CHUNK):
        ones_v[r] = one16
    _zero_zbuf(zbuf_v, DEGW)
    pltpu.sync_copy(dstr_hbm.at[wid], didx_v)
    _zero_acc_slice(zbuf_v, deg_sh, sid, sem)
    plsc.subcore_barrier()

    K = 25

    def sbody(gidx, carry):
        cps = []
        for u in range(K):
            i = gidx * K + u
            cps.append(pltpu.async_copy(ones_v, deg_sh.at[didx_v.at[i]], sem,
                                        add=True))
        for cp in cps:
            cp.wait()
        return carry

    lax.fori_loop(0, NCHUNKS // K, sbody, 0)
    plsc.subcore_barrier()
    _copy_out_slice(deg_sh, deg_hbm, cid, sid)


_sc_deg = functools.partial(
    pl.kernel,
    out_type=jax.ShapeDtypeStruct((NC, N, DEGW), f32),
    mesh=_mesh,
    scratch_types=[
        pltpu.VMEM((NCHUNKS, CHUNK), jnp.int32),
        pltpu.VMEM((CHUNK, DEGW), f32),
        pltpu.VMEM((16, DEGW), f32),
        pltpu.VMEM_SHARED((N, DEGW), f32),
        pltpu.SemaphoreType.DMA,
    ],
)(_sc_deg_body)


NBUF = 3
NWIN = 5
WCH = 25


def _sc_agg_body(g_hbm, srcr_hbm, dstr_hbm, out_hbm,
                 sidx_v, didx_v, rows_v, zbuf_v, acc_sh,
                 g0, g1, g2, s0, s1, s2, isem):
    cid = lax.axis_index("c")
    sid = lax.axis_index("s")
    wid = sid * NC + cid
    gsems = (g0, g1, g2)
    ssems = (s0, s1, s2)
    _zero_zbuf(zbuf_v, D)
    pltpu.async_copy(srcr_hbm.at[wid].at[0], sidx_v.at[0], isem)
    pltpu.async_copy(dstr_hbm.at[wid].at[0], didx_v.at[0], isem)
    _zero_acc_slice(zbuf_v, acc_sh, sid, g0)
    plsc.subcore_barrier()

    def win(w, carry):
        ws = lax.rem(w, 2)
        sx = sidx_v.at[ws]
        dx = didx_v.at[ws]
        pltpu.make_async_copy(srcr_hbm.at[wid].at[w], sx, isem).wait()
        pltpu.make_async_copy(dstr_hbm.at[wid].at[w], dx, isem).wait()

        @pl.when(w + 1 < NWIN)
        def _pfw():
            pltpu.async_copy(srcr_hbm.at[wid].at[w + 1], sidx_v.at[1 - ws],
                             isem)
            pltpu.async_copy(dstr_hbm.at[wid].at[w + 1], didx_v.at[1 - ws],
                             isem)

        for b in range(NBUF):
            pltpu.async_copy(g_hbm.at[sx.at[b]], rows_v.at[b], gsems[b])

        def body(i, c2):
            for b in range(NBUF):
                t = i * NBUF + b
                pltpu.make_async_copy(g_hbm.at[sx.at[t]], rows_v.at[b],
                                      gsems[b]).wait()
                pltpu.async_copy(rows_v.at[b], acc_sh.at[dx.at[t]],
                                 ssems[b], add=True)

                @pl.when(t + NBUF < WCH)
                def _next():
                    pltpu.make_async_copy(rows_v.at[b],
                                          acc_sh.at[dx.at[t]],
                                          ssems[b]).wait()
                    pltpu.async_copy(g_hbm.at[sx.at[t + NBUF]],
                                     rows_v.at[b], gsems[b])
            return c2

        lax.fori_loop(0, WCH // NBUF, body, 0)
        for t2 in range((WCH // NBUF) * NBUF, WCH):
            bb = t2 % NBUF
            pltpu.make_async_copy(g_hbm.at[sx.at[t2]], rows_v.at[bb],
                                  gsems[bb]).wait()
            pltpu.async_copy(rows_v.at[bb], acc_sh.at[dx.at[t2]],
                             ssems[bb], add=True)
        for t2 in range(WCH - NBUF, WCH):
            bb = t2 % NBUF
            pltpu.make_async_copy(rows_v.at[bb], acc_sh.at[dx.at[t2]],
                                  ssems[bb]).wait()
        return carry

    lax.fori_loop(0, NWIN, win, 0)
    plsc.subcore_barrier()
    _copy_out_slice(acc_sh, out_hbm, cid, sid)


_sc_agg = functools.partial(
    pl.kernel,
    out_type=jax.ShapeDtypeStruct((NC, N, D), f32),
    mesh=_mesh,
    scratch_types=[
        pltpu.VMEM((2, WCH, CHUNK), jnp.int32),
        pltpu.VMEM((2, WCH, CHUNK), jnp.int32),
        pltpu.VMEM((NBUF, CHUNK, D), f32),
        pltpu.VMEM((16, D), f32),
        pltpu.VMEM_SHARED((N, D), f32),
        pltpu.SemaphoreType.DMA,
        pltpu.SemaphoreType.DMA,
        pltpu.SemaphoreType.DMA,
        pltpu.SemaphoreType.DMA,
        pltpu.SemaphoreType.DMA,
        pltpu.SemaphoreType.DMA,
        pltpu.SemaphoreType.DMA,
    ],
)(_sc_agg_body)


MB = 1000
GRID = N // MB


def _tc_mm_body(x_ref, w_ref, u_ref):
    u_ref[...] = jnp.dot(x_ref[...], w_ref[...], preferred_element_type=f32)


_tc_mm = pl.pallas_call(
    _tc_mm_body,
    grid=(GRID,),
    in_specs=[
        pl.BlockSpec((MB, D), lambda i: (i, 0)),
        pl.BlockSpec((D, D), lambda i: (0, 0)),
    ],
    out_specs=pl.BlockSpec((MB, D), lambda i: (i, 0)),
    out_shape=jax.ShapeDtypeStruct((N, D), f32),
)


def _tc_scale_body(degs_ref, u_ref, g_ref, dinv_ref):
    deg = degs_ref[0, :, 0:1] + degs_ref[1, :, 0:1] + 1.0
    dinv = lax.rsqrt(deg)
    g_ref[...] = u_ref[...] * dinv
    dinv_ref[...] = dinv


_tc_scale = pl.pallas_call(
    _tc_scale_body,
    grid=(GRID,),
    in_specs=[
        pl.BlockSpec((2, MB, DEGW), lambda i: (0, i, 0)),
        pl.BlockSpec((MB, D), lambda i: (i, 0)),
    ],
    out_specs=[
        pl.BlockSpec((MB, D), lambda i: (i, 0)),
        pl.BlockSpec((MB, 1), lambda i: (i, 0)),
    ],
    out_shape=[
        jax.ShapeDtypeStruct((N, D), f32),
        jax.ShapeDtypeStruct((N, 1), f32),
    ],
)


def _tc_mid_body(s_ref, g_ref, dinv_ref, b_ref, w_ref, gout_ref):
    dinv = dinv_ref[...]
    z = dinv * (s_ref[0] + s_ref[1] + g_ref[...]) + b_ref[...]
    r = jnp.maximum(z, 0.0)
    gout_ref[...] = jnp.dot(r, w_ref[...], preferred_element_type=f32) * dinv


_tc_mid = pl.pallas_call(
    _tc_mid_body,
    grid=(GRID,),
    in_specs=[
        pl.BlockSpec((2, MB, D), lambda i: (0, i, 0)),
        pl.BlockSpec((MB, D), lambda i: (i, 0)),
        pl.BlockSpec((MB, 1), lambda i: (i, 0)),
        pl.BlockSpec((D,), lambda i: (0,)),
        pl.BlockSpec((D, D), lambda i: (0, 0)),
    ],
    out_specs=pl.BlockSpec((MB, D), lambda i: (i, 0)),
    out_shape=jax.ShapeDtypeStruct((N, D), f32),
)


def _tc_last_body(s_ref, g_ref, dinv_ref, b_ref, out_ref):
    z = dinv_ref[...] * (s_ref[0] + s_ref[1] + g_ref[...]) + b_ref[...]
    m = jnp.max(z, axis=-1, keepdims=True)
    e = jnp.exp(z - m)
    lse = jnp.log(jnp.sum(e, axis=-1, keepdims=True))
    out_ref[...] = z - m - lse


_tc_last = pl.pallas_call(
    _tc_last_body,
    grid=(GRID,),
    in_specs=[
        pl.BlockSpec((2, MB, D), lambda i: (0, i, 0)),
        pl.BlockSpec((MB, D), lambda i: (i, 0)),
        pl.BlockSpec((MB, 1), lambda i: (i, 0)),
        pl.BlockSpec((D,), lambda i: (0,)),
    ],
    out_specs=pl.BlockSpec((MB, D), lambda i: (i, 0)),
    out_shape=jax.ShapeDtypeStruct((N, D), f32),
)


def kernel(x, edge_index, W1, b1, W2, b2, W3, b3):
    src = edge_index[0]
    dst = edge_index[1]
    srcr = src.reshape(NW, NWIN, WCH, CHUNK)
    dstr = dst.reshape(NW, NWIN, WCH, CHUNK)

    degs = _sc_deg(dst.reshape(NW, NCHUNKS, CHUNK))

    u1 = _tc_mm(x, W1)
    g1, dinv = _tc_scale(degs, u1)
    s1 = _sc_agg(g1, srcr, dstr)
    g2 = _tc_mid(s1, g1, dinv, b1, W2)
    s2 = _sc_agg(g2, srcr, dstr)
    g3 = _tc_mid(s2, g2, dinv, b2, W3)
    s3 = _sc_agg(g3, srcr, dstr)
    return _tc_last(s3, g3, dinv, b3)


# speedup vs baseline: 31.0436x; 1.0211x over previous
import functools

import jax
import jax.numpy as jnp
from jax import lax
from jax.experimental import pallas as pl
from jax.experimental.pallas import tpu as pltpu
from jax.experimental.pallas import tpu_sc as plsc

f32 = jnp.float32

N = 10000
D = 128
E = 320000

NC = 2
NS = 16
NW = NC * NS
EPW = E // NW
CHUNK = 80
NCHUNKS = EPW // CHUNK
ZR = 624
TAIL = N - NS * ZR
DEGW = 16

_mesh = plsc.VectorSubcoreMesh(core_axis_name="c", subcore_axis_name="s")


def _zero_zbuf(zbuf_v, width):
    z16 = jnp.zeros((16,), f32)
    for r in range(16):
        if width == 16:
            zbuf_v[r] = z16
        else:
            for c in range(width // 16):
                zbuf_v[r, pl.ds(c * 16, 16)] = z16


def _zero_acc_slice(zbuf_v, acc_sh, sid, zsem):
    zbase = sid * ZR

    def zbody(c, carry):
        pltpu.async_copy(zbuf_v, acc_sh.at[pl.ds(zbase + c * 16, 16)], zsem)
        return carry

    lax.fori_loop(0, ZR // 16, zbody, 0)

    @pl.when(sid == NS - 1)
    def _ztail():
        pltpu.async_copy(zbuf_v, acc_sh.at[pl.ds(NS * ZR, TAIL)], zsem)

    def zdrain(c, carry):
        pltpu.make_async_copy(zbuf_v, acc_sh.at[pl.ds(zbase, 16)], zsem).wait()
        return carry

    lax.fori_loop(0, ZR // 16, zdrain, 0)

    @pl.when(sid == NS - 1)
    def _zdtail():
        pltpu.make_async_copy(zbuf_v, acc_sh.at[pl.ds(zbase, TAIL)],
                              zsem).wait()


def _copy_out_slice(acc_sh, out_hbm, cid, sid):
    zbase = sid * ZR
    pltpu.sync_copy(acc_sh.at[pl.ds(zbase, ZR)],
                    out_hbm.at[cid].at[pl.ds(zbase, ZR)])

    @pl.when(sid == NS - 1)
    def _ctail():
        pltpu.sync_copy(acc_sh.at[pl.ds(NS * ZR, TAIL)],
                        out_hbm.at[cid].at[pl.ds(NS * ZR, TAIL)])


def _sc_deg_body(dstr_hbm, deg_hbm, didx_v, ones_v, zbuf_v, deg_sh, sem):
    cid = lax.axis_index("c")
    sid = lax.axis_index("s")
    wid = sid * NC + cid
    one16 = jnp.ones((16,), f32)
    for r in range(CHUNK):
        ones_v[r] = one16
    _zero_zbuf(zbuf_v, DEGW)
    pltpu.sync_copy(dstr_hbm.at[wid], didx_v)
    _zero_acc_slice(zbuf_v, deg_sh, sid, sem)
    plsc.subcore_barrier()

    K = 25

    def sbody(gidx, carry):
        cps = []
        for u in range(K):
            i = gidx * K + u
            cps.append(pltpu.async_copy(ones_v, deg_sh.at[didx_v.at[i]], sem,
                                        add=True))
        for cp in cps:
            cp.wait()
        return carry

    lax.fori_loop(0, NCHUNKS // K, sbody, 0)
    plsc.subcore_barrier()
    _copy_out_slice(deg_sh, deg_hbm, cid, sid)


_sc_deg = functools.partial(
    pl.kernel,
    out_type=jax.ShapeDtypeStruct((NC, N, DEGW), f32),
    mesh=_mesh,
    scratch_types=[
        pltpu.VMEM((NCHUNKS, CHUNK), jnp.int32),
        pltpu.VMEM((CHUNK, DEGW), f32),
        pltpu.VMEM((16, DEGW), f32),
        pltpu.VMEM_SHARED((N, DEGW), f32),
        pltpu.SemaphoreType.DMA,
    ],
)(_sc_deg_body)


NBUF = 3
NWIN = 5
WCH = 25


def _sc_agg_body(g_hbm, srcr_hbm, dstr_hbm, out_hbm,
                 sidx_v, didx_v, rows_v, zbuf_v, acc_sh,
                 g0, g1, g2, s0, s1, s2, isem):
    cid = lax.axis_index("c")
    sid = lax.axis_index("s")
    wid = sid * NC + cid
    gsems = (g0, g1, g2)
    ssems = (s0, s1, s2)
    _zero_zbuf(zbuf_v, D)
    pltpu.async_copy(srcr_hbm.at[wid].at[0], sidx_v.at[0], isem)
    pltpu.async_copy(dstr_hbm.at[wid].at[0], didx_v.at[0], isem)
    _zero_acc_slice(zbuf_v, acc_sh, sid, g0)
    plsc.subcore_barrier()

    def win(w, carry):
        ws = lax.rem(w, 2)
        sx = sidx_v.at[ws]
        dx = didx_v.at[ws]
        pltpu.make_async_copy(srcr_hbm.at[wid].at[w], sx, isem).wait()
        pltpu.make_async_copy(dstr_hbm.at[wid].at[w], dx, isem).wait()

        @pl.when(w + 1 < NWIN)
        def _pfw():
            pltpu.async_copy(srcr_hbm.at[wid].at[w + 1], sidx_v.at[1 - ws],
                             isem)
            pltpu.async_copy(dstr_hbm.at[wid].at[w + 1], didx_v.at[1 - ws],
                             isem)

        for b in range(NBUF):
            pltpu.async_copy(g_hbm.at[sx.at[b]], rows_v.at[b], gsems[b])

        def body(i, c2):
            for b in range(NBUF):
                t = i * NBUF + b
                pltpu.make_async_copy(g_hbm.at[sx.at[t]], rows_v.at[b],
                                      gsems[b]).wait()
                pltpu.async_copy(rows_v.at[b], acc_sh.at[dx.at[t]],
                                 ssems[b], add=True)

                @pl.when(t + NBUF < WCH)
                def _next():
                    pltpu.make_async_copy(rows_v.at[b],
                                          acc_sh.at[dx.at[t]],
                                          ssems[b]).wait()
                    pltpu.async_copy(g_hbm.at[sx.at[t + NBUF]],
                                     rows_v.at[b], gsems[b])
            return c2

        lax.fori_loop(0, WCH // NBUF, body, 0)
        for t2 in range((WCH // NBUF) * NBUF, WCH):
            bb = t2 % NBUF
            pltpu.make_async_copy(g_hbm.at[sx.at[t2]], rows_v.at[bb],
                                  gsems[bb]).wait()
            pltpu.async_copy(rows_v.at[bb], acc_sh.at[dx.at[t2]],
                             ssems[bb], add=True)
        for t2 in range(WCH - NBUF, WCH):
            bb = t2 % NBUF
            pltpu.make_async_copy(rows_v.at[bb], acc_sh.at[dx.at[t2]],
                                  ssems[bb]).wait()
        return carry

    lax.fori_loop(0, NWIN, win, 0)
    plsc.subcore_barrier()
    _copy_out_slice(acc_sh, out_hbm, cid, sid)


_sc_agg = functools.partial(
    pl.kernel,
    out_type=jax.ShapeDtypeStruct((NC, N, D), f32),
    mesh=_mesh,
    scratch_types=[
        pltpu.VMEM((2, WCH, CHUNK), jnp.int32),
        pltpu.VMEM((2, WCH, CHUNK), jnp.int32),
        pltpu.VMEM((NBUF, CHUNK, D), f32),
        pltpu.VMEM((16, D), f32),
        pltpu.VMEM_SHARED((N, D), f32),
        pltpu.SemaphoreType.DMA,
        pltpu.SemaphoreType.DMA,
        pltpu.SemaphoreType.DMA,
        pltpu.SemaphoreType.DMA,
        pltpu.SemaphoreType.DMA,
        pltpu.SemaphoreType.DMA,
        pltpu.SemaphoreType.DMA,
    ],
)(_sc_agg_body)


MB = 2000
GRID = N // MB


def _tc_first_body(degs_ref, x_ref, w_ref, g_ref, dinv_ref):
    deg = degs_ref[0, :, 0:1] + degs_ref[1, :, 0:1] + 1.0
    dinv = lax.rsqrt(deg)
    h = jnp.dot(x_ref[...], w_ref[...], preferred_element_type=f32)
    g_ref[...] = h * dinv
    dinv_ref[...] = dinv


_tc_first = pl.pallas_call(
    _tc_first_body,
    grid=(GRID,),
    in_specs=[
        pl.BlockSpec((2, MB, DEGW), lambda i: (0, i, 0)),
        pl.BlockSpec((MB, D), lambda i: (i, 0)),
        pl.BlockSpec((D, D), lambda i: (0, 0)),
    ],
    out_specs=[
        pl.BlockSpec((MB, D), lambda i: (i, 0)),
        pl.BlockSpec((MB, 1), lambda i: (i, 0)),
    ],
    out_shape=[
        jax.ShapeDtypeStruct((N, D), f32),
        jax.ShapeDtypeStruct((N, 1), f32),
    ],
)


def _tc_mid_body(s_ref, g_ref, dinv_ref, b_ref, w_ref, gout_ref):
    dinv = dinv_ref[...]
    z = dinv * (s_ref[0] + s_ref[1] + g_ref[...]) + b_ref[...]
    r = jnp.maximum(z, 0.0)
    gout_ref[...] = jnp.dot(r, w_ref[...], preferred_element_type=f32) * dinv


_tc_mid = pl.pallas_call(
    _tc_mid_body,
    grid=(GRID,),
    in_specs=[
        pl.BlockSpec((2, MB, D), lambda i: (0, i, 0)),
        pl.BlockSpec((MB, D), lambda i: (i, 0)),
        pl.BlockSpec((MB, 1), lambda i: (i, 0)),
        pl.BlockSpec((D,), lambda i: (0,)),
        pl.BlockSpec((D, D), lambda i: (0, 0)),
    ],
    out_specs=pl.BlockSpec((MB, D), lambda i: (i, 0)),
    out_shape=jax.ShapeDtypeStruct((N, D), f32),
)


def _tc_last_body(s_ref, g_ref, dinv_ref, b_ref, out_ref):
    z = dinv_ref[...] * (s_ref[0] + s_ref[1] + g_ref[...]) + b_ref[...]
    m = jnp.max(z, axis=-1, keepdims=True)
    e = jnp.exp(z - m)
    lse = jnp.log(jnp.sum(e, axis=-1, keepdims=True))
    out_ref[...] = z - m - lse


_tc_last = pl.pallas_call(
    _tc_last_body,
    grid=(GRID,),
    in_specs=[
        pl.BlockSpec((2, MB, D), lambda i: (0, i, 0)),
        pl.BlockSpec((MB, D), lambda i: (i, 0)),
        pl.BlockSpec((MB, 1), lambda i: (i, 0)),
        pl.BlockSpec((D,), lambda i: (0,)),
    ],
    out_specs=pl.BlockSpec((MB, D), lambda i: (i, 0)),
    out_shape=jax.ShapeDtypeStruct((N, D), f32),
)


def kernel(x, edge_index, W1, b1, W2, b2, W3, b3):
    src = edge_index[0]
    dst = edge_index[1]
    srcr = src.reshape(NW, NWIN, WCH, CHUNK)
    dstr = dst.reshape(NW, NWIN, WCH, CHUNK)

    degs = _sc_deg(dst.reshape(NW, NCHUNKS, CHUNK))

    g1, dinv = _tc_first(degs, x, W1)
    s1 = _sc_agg(g1, srcr, dstr)
    g2 = _tc_mid(s1, g1, dinv, b1, W2)
    s2 = _sc_agg(g2, srcr, dstr)
    g3 = _tc_mid(s2, g2, dinv, b2, W3)
    s3 = _sc_agg(g3, srcr, dstr)
    return _tc_last(s3, g3, dinv, b3)


# speedup vs baseline: 31.2208x; 1.0057x over previous
import functools

import jax
import jax.numpy as jnp
from jax import lax
from jax.experimental import pallas as pl
from jax.experimental.pallas import tpu as pltpu
from jax.experimental.pallas import tpu_sc as plsc

f32 = jnp.float32

N = 10000
D = 128
E = 320000

NC = 2
NS = 16
NW = NC * NS
EPW = E // NW
CHUNK = 80
NCHUNKS = EPW // CHUNK
ZR = 624
TAIL = N - NS * ZR
DEGW = 16

_mesh = plsc.VectorSubcoreMesh(core_axis_name="c", subcore_axis_name="s")


def _zero_zbuf(zbuf_v, width):
    z16 = jnp.zeros((16,), f32)
    for r in range(16):
        if width == 16:
            zbuf_v[r] = z16
        else:
            for c in range(width // 16):
                zbuf_v[r, pl.ds(c * 16, 16)] = z16


def _zero_acc_slice(zbuf_v, acc_sh, sid, zsem):
    zbase = sid * ZR

    def zbody(c, carry):
        pltpu.async_copy(zbuf_v, acc_sh.at[pl.ds(zbase + c * 16, 16)], zsem)
        return carry

    lax.fori_loop(0, ZR // 16, zbody, 0)

    @pl.when(sid == NS - 1)
    def _ztail():
        pltpu.async_copy(zbuf_v, acc_sh.at[pl.ds(NS * ZR, TAIL)], zsem)

    def zdrain(c, carry):
        pltpu.make_async_copy(zbuf_v, acc_sh.at[pl.ds(zbase, 16)], zsem).wait()
        return carry

    lax.fori_loop(0, ZR // 16, zdrain, 0)

    @pl.when(sid == NS - 1)
    def _zdtail():
        pltpu.make_async_copy(zbuf_v, acc_sh.at[pl.ds(zbase, TAIL)],
                              zsem).wait()


def _copy_out_slice(acc_sh, out_hbm, cid, sid):
    zbase = sid * ZR
    pltpu.sync_copy(acc_sh.at[pl.ds(zbase, ZR)],
                    out_hbm.at[cid].at[pl.ds(zbase, ZR)])

    @pl.when(sid == NS - 1)
    def _ctail():
        pltpu.sync_copy(acc_sh.at[pl.ds(NS * ZR, TAIL)],
                        out_hbm.at[cid].at[pl.ds(NS * ZR, TAIL)])


def _sc_deg_body(dstr_hbm, deg_hbm, didx_v, ones_v, zbuf_v, deg_sh, sem):
    cid = lax.axis_index("c")
    sid = lax.axis_index("s")
    wid = sid * NC + cid
    one16 = jnp.ones((16,), f32)
    for r in range(CHUNK):
        ones_v[r] = one16
    _zero_zbuf(zbuf_v, DEGW)
    pltpu.sync_copy(dstr_hbm.at[wid], didx_v)
    _zero_acc_slice(zbuf_v, deg_sh, sid, sem)
    plsc.subcore_barrier()

    K = 25

    def sbody(gidx, carry):
        cps = []
        for u in range(K):
            i = gidx * K + u
            cps.append(pltpu.async_copy(ones_v, deg_sh.at[didx_v.at[i]], sem,
                                        add=True))
        for cp in cps:
            cp.wait()
        return carry

    lax.fori_loop(0, NCHUNKS // K, sbody, 0)
    plsc.subcore_barrier()
    _copy_out_slice(deg_sh, deg_hbm, cid, sid)


_sc_deg = functools.partial(
    pl.kernel,
    out_type=jax.ShapeDtypeStruct((NC, N, DEGW), f32),
    mesh=_mesh,
    scratch_types=[
        pltpu.VMEM((NCHUNKS, CHUNK), jnp.int32),
        pltpu.VMEM((CHUNK, DEGW), f32),
        pltpu.VMEM((16, DEGW), f32),
        pltpu.VMEM_SHARED((N, DEGW), f32),
        pltpu.SemaphoreType.DMA,
    ],
)(_sc_deg_body)


NBUF = 3
NWIN = 5
WCH = 25


def _sc_agg_body(g_hbm, srcr_hbm, dstr_hbm, out_hbm,
                 sidx_v, didx_v, rows_v, zbuf_v, acc_sh,
                 g0, g1, g2, s0, s1, s2, isem):
    cid = lax.axis_index("c")
    sid = lax.axis_index("s")
    wid = sid * NC + cid
    gsems = (g0, g1, g2)
    ssems = (s0, s1, s2)
    _zero_zbuf(zbuf_v, D)
    pltpu.async_copy(srcr_hbm.at[wid].at[0], sidx_v.at[0], isem)
    pltpu.async_copy(dstr_hbm.at[wid].at[0], didx_v.at[0], isem)
    _zero_acc_slice(zbuf_v, acc_sh, sid, g0)
    plsc.subcore_barrier()

    def win(w, carry):
        ws = lax.rem(w, 2)
        sx = sidx_v.at[ws]
        dx = didx_v.at[ws]
        pltpu.make_async_copy(srcr_hbm.at[wid].at[w], sx, isem).wait()
        pltpu.make_async_copy(dstr_hbm.at[wid].at[w], dx, isem).wait()

        @pl.when(w + 1 < NWIN)
        def _pfw():
            pltpu.async_copy(srcr_hbm.at[wid].at[w + 1], sidx_v.at[1 - ws],
                             isem)
            pltpu.async_copy(dstr_hbm.at[wid].at[w + 1], didx_v.at[1 - ws],
                             isem)

        for b in range(NBUF):
            pltpu.async_copy(g_hbm.at[sx.at[b]], rows_v.at[b], gsems[b])

        def body(i, c2):
            for b in range(NBUF):
                t = i * NBUF + b
                pltpu.make_async_copy(g_hbm.at[sx.at[t]], rows_v.at[b],
                                      gsems[b]).wait()
                pltpu.async_copy(rows_v.at[b], acc_sh.at[dx.at[t]],
                                 ssems[b], add=True)

                @pl.when(t + NBUF < WCH)
                def _next():
                    pltpu.make_async_copy(rows_v.at[b],
                                          acc_sh.at[dx.at[t]],
                                          ssems[b]).wait()
                    pltpu.async_copy(g_hbm.at[sx.at[t + NBUF]],
                                     rows_v.at[b], gsems[b])
            return c2

        lax.fori_loop(0, WCH // NBUF, body, 0)
        for t2 in range((WCH // NBUF) * NBUF, WCH):
            bb = t2 % NBUF
            pltpu.make_async_copy(g_hbm.at[sx.at[t2]], rows_v.at[bb],
                                  gsems[bb]).wait()
            pltpu.async_copy(rows_v.at[bb], acc_sh.at[dx.at[t2]],
                             ssems[bb], add=True)
        for t2 in range(WCH - NBUF, WCH):
            bb = t2 % NBUF
            pltpu.make_async_copy(rows_v.at[bb], acc_sh.at[dx.at[t2]],
                                  ssems[bb]).wait()
        return carry

    lax.fori_loop(0, NWIN, win, 0)
    plsc.subcore_barrier()
    _copy_out_slice(acc_sh, out_hbm, cid, sid)


_sc_agg = functools.partial(
    pl.kernel,
    out_type=jax.ShapeDtypeStruct((NC, N, D), f32),
    mesh=_mesh,
    scratch_types=[
        pltpu.VMEM((2, WCH, CHUNK), jnp.int32),
        pltpu.VMEM((2, WCH, CHUNK), jnp.int32),
        pltpu.VMEM((NBUF, CHUNK, D), f32),
        pltpu.VMEM((16, D), f32),
        pltpu.VMEM_SHARED((N, D), f32),
        pltpu.SemaphoreType.DMA,
        pltpu.SemaphoreType.DMA,
        pltpu.SemaphoreType.DMA,
        pltpu.SemaphoreType.DMA,
        pltpu.SemaphoreType.DMA,
        pltpu.SemaphoreType.DMA,
        pltpu.SemaphoreType.DMA,
    ],
)(_sc_agg_body)


MB = 5000
GRID = N // MB


def _tc_first_body(degs_ref, x_ref, w_ref, g_ref, dinv_ref):
    deg = degs_ref[0, :, 0:1] + degs_ref[1, :, 0:1] + 1.0
    dinv = lax.rsqrt(deg)
    h = jnp.dot(x_ref[...], w_ref[...], preferred_element_type=f32)
    g_ref[...] = h * dinv
    dinv_ref[...] = dinv


_tc_first = pl.pallas_call(
    _tc_first_body,
    grid=(GRID,),
    in_specs=[
        pl.BlockSpec((2, MB, DEGW), lambda i: (0, i, 0)),
        pl.BlockSpec((MB, D), lambda i: (i, 0)),
        pl.BlockSpec((D, D), lambda i: (0, 0)),
    ],
    out_specs=[
        pl.BlockSpec((MB, D), lambda i: (i, 0)),
        pl.BlockSpec((MB, 1), lambda i: (i, 0)),
    ],
    out_shape=[
        jax.ShapeDtypeStruct((N, D), f32),
        jax.ShapeDtypeStruct((N, 1), f32),
    ],
)


def _tc_mid_body(s_ref, g_ref, dinv_ref, b_ref, w_ref, gout_ref):
    dinv = dinv_ref[...]
    z = dinv * (s_ref[0] + s_ref[1] + g_ref[...]) + b_ref[...]
    r = jnp.maximum(z, 0.0)
    gout_ref[...] = jnp.dot(r, w_ref[...], preferred_element_type=f32) * dinv


_tc_mid = pl.pallas_call(
    _tc_mid_body,
    grid=(GRID,),
    in_specs=[
        pl.BlockSpec((2, MB, D), lambda i: (0, i, 0)),
        pl.BlockSpec((MB, D), lambda i: (i, 0)),
        pl.BlockSpec((MB, 1), lambda i: (i, 0)),
        pl.BlockSpec((D,), lambda i: (0,)),
        pl.BlockSpec((D, D), lambda i: (0, 0)),
    ],
    out_specs=pl.BlockSpec((MB, D), lambda i: (i, 0)),
    out_shape=jax.ShapeDtypeStruct((N, D), f32),
)


def _tc_last_body(s_ref, g_ref, dinv_ref, b_ref, out_ref):
    z = dinv_ref[...] * (s_ref[0] + s_ref[1] + g_ref[...]) + b_ref[...]
    m = jnp.max(z, axis=-1, keepdims=True)
    e = jnp.exp(z - m)
    lse = jnp.log(jnp.sum(e, axis=-1, keepdims=True))
    out_ref[...] = z - m - lse


_tc_last = pl.pallas_call(
    _tc_last_body,
    grid=(GRID,),
    in_specs=[
        pl.BlockSpec((2, MB, D), lambda i: (0, i, 0)),
        pl.BlockSpec((MB, D), lambda i: (i, 0)),
        pl.BlockSpec((MB, 1), lambda i: (i, 0)),
        pl.BlockSpec((D,), lambda i: (0,)),
    ],
    out_specs=pl.BlockSpec((MB, D), lambda i: (i, 0)),
    out_shape=jax.ShapeDtypeStruct((N, D), f32),
)


def kernel(x, edge_index, W1, b1, W2, b2, W3, b3):
    src = edge_index[0]
    dst = edge_index[1]
    srcr = src.reshape(NW, NWIN, WCH, CHUNK)
    dstr = dst.reshape(NW, NWIN, WCH, CHUNK)

    degs = _sc_deg(dst.reshape(NW, NCHUNKS, CHUNK))

    g1, dinv = _tc_first(degs, x, W1)
    s1 = _sc_agg(g1, srcr, dstr)
    g2 = _tc_mid(s1, g1, dinv, b1, W2)
    s2 = _sc_agg(g2, srcr, dstr)
    g3 = _tc_mid(s2, g2, dinv, b2, W3)
    s3 = _sc_agg(g3, srcr, dstr)
    return _tc_last(s3, g3, dinv, b3)
